# baseline (speedup 1.0000x reference)
import jax
import jax.numpy as jnp
from jax.experimental import pallas as pl
from jax.experimental.pallas import tpu as pltpu

T = 0.07
K51 = 51.0
N_BISECT = 2
N_GBISECT = 12
GW = 25
B = 1024
D = 128
M = 100000
MT = 2000
N_TILES = M // MT
NG_TILE = MT // GW
NG = M // GW
CAP = 0.05
N_PHASES = 2 + N_BISECT


def _phase_kernel(pi_ref, points_ref, mb_ref, out1_ref, out2_ref,
                  pn_ref, vpos_ref, gmax_ref, lo_ref, hi_ref,
                  cnt_ref, sa_ref, sb_ref, sl_ref):
    p = pl.program_id(0)
    t = pl.program_id(1)

    @pl.when(jnp.logical_and(p == 0, t == 0))
    def _init():
        pts = points_ref[...]
        nrm = jnp.sqrt(jnp.sum(pts * pts, axis=1, keepdims=True)) + 1e-12
        pn_ref[...] = pts / nrm
        vpos_ref[...] = jnp.zeros((B, 1), jnp.float32)

    d = jax.lax.dot_general(pn_ref[...], mb_ref[...],
                            (((1,), (1,)), ((), ())),
                            preferred_element_type=jnp.float32)

    @pl.when(p == 0)
    def _max_vpos_gmax():
        jg = t * MT + jax.lax.broadcasted_iota(jnp.int32, (B, MT), 1)
        sel = jnp.where(jg == pi_ref[...], d, 0.0)
        vpos_ref[...] += jnp.sum(sel, axis=1, keepdims=True)
        gm = jnp.concatenate(
            [jnp.max(d[:, g * GW:(g + 1) * GW], axis=1, keepdims=True)
             for g in range(NG_TILE)], axis=1)
        gmax_ref[pl.ds(t * NG_TILE, NG_TILE), :] = jnp.transpose(gm, (1, 0))

        @pl.when(t == N_TILES - 1)
        def _start():
            gmax = gmax_ref[...]
            rmax = jnp.max(gmax, axis=0, keepdims=True)
            glo = jnp.full((1, B), 2.0, jnp.float32)
            ghi = rmax
            for _ in range(N_GBISECT):
                mid = 0.5 * (glo + ghi)
                cg = jnp.sum((gmax > mid).astype(jnp.float32),
                             axis=0, keepdims=True)
                ge = cg >= K51
                glo = jnp.where(ge, mid, glo)
                ghi = jnp.where(ge, ghi, mid)
            lo_t = jnp.minimum(glo, rmax - 1e-3)
            hi_t = jnp.minimum(rmax, glo + CAP)
            lo_ref[...] = jnp.transpose(lo_t, (1, 0))
            hi_ref[...] = jnp.transpose(hi_t, (1, 0))

    @pl.when(jnp.logical_and(p >= 1, p <= N_BISECT))
    def _bisect():
        @pl.when(t == 0)
        def _zero():
            cnt_ref[...] = jnp.zeros((B, 1), jnp.float32)

        mid = 0.5 * (lo_ref[...] + hi_ref[...])
        cnt_ref[...] += jnp.sum((d > mid).astype(jnp.float32),
                                axis=1, keepdims=True)

        @pl.when(t == N_TILES - 1)
        def _update():
            mid2 = 0.5 * (lo_ref[...] + hi_ref[...])
            ge = cnt_ref[...] >= K51
            lo_ref[...] = jnp.where(ge, mid2, lo_ref[...])
            hi_ref[...] = jnp.where(ge, hi_ref[...], mid2)

    @pl.when(p == N_PHASES - 1)
    def _sums():
        @pl.when(t == 0)
        def _zero():
            sa_ref[...] = jnp.zeros((B, 2), jnp.float32)
            sb_ref[...] = jnp.zeros((B, 2), jnp.float32)
            sl_ref[...] = jnp.zeros((B, 1), jnp.float32)

        lo = lo_ref[...]
        hi = hi_ref[...]
        e = jnp.exp(d * (1.0 / T))
        above = d > hi
        gtlo = d > lo
        band = jnp.logical_and(gtlo, ~above)
        onez = jnp.ones((B, MT), jnp.float32)
        sa_ref[...] += jnp.concatenate(
            [jnp.sum(jnp.where(above, e, 0.0), axis=1, keepdims=True),
             jnp.sum(jnp.where(above, onez, 0.0), axis=1, keepdims=True)],
            axis=1)
        sb_ref[...] += jnp.concatenate(
            [jnp.sum(jnp.where(band, e, 0.0), axis=1, keepdims=True),
             jnp.sum(jnp.where(band, onez, 0.0), axis=1, keepdims=True)],
            axis=1)
        sl_ref[...] += jnp.sum(jnp.where(gtlo, 0.0, e), axis=1, keepdims=True)

        @pl.when(t == N_TILES - 1)
        def _loss():
            vpos = vpos_ref[...]
            pos_sim = jnp.exp(vpos * (1.0 / T))
            s_above = sa_ref[:, 0:1]
            chi = sa_ref[:, 1:2]
            s_band = sb_ref[:, 0:1]
            n_band = sb_ref[:, 1:2]
            s_below = sl_ref[...]
            denom = s_above + s_band + s_below
            take = jnp.clip(K51 - chi, 0.0, n_band)
            avg_band = s_band / jnp.maximum(n_band, 1.0)
            s_rest = s_below + (n_band - take) * avg_band
            has_self = (vpos > lo_ref[...]).astype(jnp.float32)
            eps = (s_rest + has_self * pos_sim) / denom
            loss2 = -jnp.mean(jnp.log1p(-eps))
            loss1 = -jnp.mean(jnp.log(pos_sim / denom + 1e-7))
            out1_ref[...] = jnp.full((1, 1), loss1, jnp.float32)
            out2_ref[...] = jnp.full((1, 1), loss2, jnp.float32)


@jax.jit
def _run(points, point_indices, memory_bank):
    pi2 = point_indices.reshape(B, 1).astype(jnp.int32)
    out1, out2 = pl.pallas_call(
        _phase_kernel,
        grid=(N_PHASES, N_TILES),
        in_specs=[
            pl.BlockSpec((B, 1), lambda p, t: (0, 0)),
            pl.BlockSpec((B, D), lambda p, t: (0, 0)),
            pl.BlockSpec((MT, D), lambda p, t: (t, 0)),
        ],
        out_specs=[
            pl.BlockSpec((1, 1), lambda p, t: (0, 0)),
            pl.BlockSpec((1, 1), lambda p, t: (0, 0)),
        ],
        out_shape=[
            jax.ShapeDtypeStruct((1, 1), jnp.float32),
            jax.ShapeDtypeStruct((1, 1), jnp.float32),
        ],
        scratch_shapes=[
            pltpu.VMEM((B, D), jnp.float32),
            pltpu.VMEM((B, 1), jnp.float32),
            pltpu.VMEM((NG, B), jnp.float32),
            pltpu.VMEM((B, 1), jnp.float32),
            pltpu.VMEM((B, 1), jnp.float32),
            pltpu.VMEM((B, 1), jnp.float32),
            pltpu.VMEM((B, 2), jnp.float32),
            pltpu.VMEM((B, 2), jnp.float32),
            pltpu.VMEM((B, 1), jnp.float32),
        ],
    )(pi2, points, memory_bank)
    return out1[0, 0], out2[0, 0]


def kernel(points, point_indices, memory_bank, first):
    loss1, loss2 = _run(points, point_indices, memory_bank)
    loss2 = loss2 + (jnp.asarray(first) * 0).astype(loss2.dtype)
    return (loss1, loss2)


# speedup vs baseline: 2.8054x; 2.8054x over previous
import jax
import jax.numpy as jnp
from jax.experimental import pallas as pl
from jax.experimental.pallas import tpu as pltpu

T = 0.07
K51 = 51.0
N_BISECT = 3
N_GBISECT = 12
GW = 125
B = 1024
D = 128
M = 100000
MT = 2000
N_TILES = M // MT
NG_TILE = MT // GW
NG = M // GW
CAP = 0.1
N_PHASES = 2 + N_BISECT


def _phase_kernel(pi_ref, points_ref, mb_ref, out1_ref, out2_ref,
                  pn_ref, vpos_ref, gmax_ref, lo_ref, hi_ref,
                  cnt_ref, sa_ref, sb_ref, sl_ref):
    p = pl.program_id(0)
    t = pl.program_id(1)

    @pl.when(jnp.logical_and(p == 0, t == 0))
    def _init():
        pts = points_ref[...]
        nrm = jnp.sqrt(jnp.sum(pts * pts, axis=1, keepdims=True)) + 1e-12
        pn_ref[...] = pts / nrm
        vpos_ref[...] = jnp.zeros((B, 1), jnp.float32)

    d = jax.lax.dot_general(pn_ref[...], mb_ref[...],
                            (((1,), (1,)), ((), ())),
                            preferred_element_type=jnp.float32)

    @pl.when(p == 0)
    def _max_vpos_gmax():
        jg = t * MT + jax.lax.broadcasted_iota(jnp.int32, (B, MT), 1)
        sel = jnp.where(jg == pi_ref[...], d, 0.0)
        vpos_ref[...] += jnp.sum(sel, axis=1, keepdims=True)
        gm = jnp.concatenate(
            [jnp.max(d[:, g * GW:(g + 1) * GW], axis=1, keepdims=True)
             for g in range(NG_TILE)], axis=1)
        gmax_ref[pl.ds(t * NG_TILE, NG_TILE), :] = jnp.transpose(gm, (1, 0))

        @pl.when(t == N_TILES - 1)
        def _start():
            gmax = gmax_ref[...]
            rmax = jnp.max(gmax, axis=0, keepdims=True)
            glo = jnp.full((1, B), 2.0, jnp.float32)
            ghi = rmax
            for _ in range(N_GBISECT):
                mid = 0.5 * (glo + ghi)
                cg = jnp.sum((gmax > mid).astype(jnp.float32),
                             axis=0, keepdims=True)
                ge = cg >= K51
                glo = jnp.where(ge, mid, glo)
                ghi = jnp.where(ge, ghi, mid)
            lo_t = jnp.minimum(glo, rmax - 1e-3)
            hi_t = jnp.minimum(rmax, glo + CAP)
            lo_ref[...] = jnp.transpose(lo_t, (1, 0))
            hi_ref[...] = jnp.transpose(hi_t, (1, 0))

    @pl.when(jnp.logical_and(p >= 1, p <= N_BISECT))
    def _bisect():
        @pl.when(t == 0)
        def _zero():
            cnt_ref[...] = jnp.zeros((B, 1), jnp.float32)

        mid = 0.5 * (lo_ref[...] + hi_ref[...])
        cnt_ref[...] += jnp.sum((d > mid).astype(jnp.float32),
                                axis=1, keepdims=True)

        @pl.when(t == N_TILES - 1)
        def _update():
            mid2 = 0.5 * (lo_ref[...] + hi_ref[...])
            ge = cnt_ref[...] >= K51
            lo_ref[...] = jnp.where(ge, mid2, lo_ref[...])
            hi_ref[...] = jnp.where(ge, hi_ref[...], mid2)

    @pl.when(p == N_PHASES - 1)
    def _sums():
        @pl.when(t == 0)
        def _zero():
            sa_ref[...] = jnp.zeros((B, 2), jnp.float32)
            sb_ref[...] = jnp.zeros((B, 2), jnp.float32)
            sl_ref[...] = jnp.zeros((B, 1), jnp.float32)

        lo = lo_ref[...]
        hi = hi_ref[...]
        e = jnp.exp(d * (1.0 / T))
        above = d > hi
        gtlo = d > lo
        band = jnp.logical_and(gtlo, ~above)
        onez = jnp.ones((B, MT), jnp.float32)
        sa_ref[...] += jnp.concatenate(
            [jnp.sum(jnp.where(above, e, 0.0), axis=1, keepdims=True),
             jnp.sum(jnp.where(above, onez, 0.0), axis=1, keepdims=True)],
            axis=1)
        sb_ref[...] += jnp.concatenate(
            [jnp.sum(jnp.where(band, e, 0.0), axis=1, keepdims=True),
             jnp.sum(jnp.where(band, onez, 0.0), axis=1, keepdims=True)],
            axis=1)
        sl_ref[...] += jnp.sum(jnp.where(gtlo, 0.0, e), axis=1, keepdims=True)

        @pl.when(t == N_TILES - 1)
        def _loss():
            vpos = vpos_ref[...]
            pos_sim = jnp.exp(vpos * (1.0 / T))
            s_above = sa_ref[:, 0:1]
            chi = sa_ref[:, 1:2]
            s_band = sb_ref[:, 0:1]
            n_band = sb_ref[:, 1:2]
            s_below = sl_ref[...]
            denom = s_above + s_band + s_below
            take = jnp.clip(K51 - chi, 0.0, n_band)
            avg_band = s_band / jnp.maximum(n_band, 1.0)
            s_rest = s_below + (n_band - take) * avg_band
            has_self = (vpos > lo_ref[...]).astype(jnp.float32)
            eps = (s_rest + has_self * pos_sim) / denom
            loss2 = -jnp.mean(jnp.log1p(-eps))
            loss1 = -jnp.mean(jnp.log(pos_sim / denom + 1e-7))
            out1_ref[...] = jnp.full((1, 1), loss1, jnp.float32)
            out2_ref[...] = jnp.full((1, 1), loss2, jnp.float32)


@jax.jit
def _run(points, point_indices, memory_bank):
    pi2 = point_indices.reshape(B, 1).astype(jnp.int32)
    out1, out2 = pl.pallas_call(
        _phase_kernel,
        grid=(N_PHASES, N_TILES),
        in_specs=[
            pl.BlockSpec((B, 1), lambda p, t: (0, 0)),
            pl.BlockSpec((B, D), lambda p, t: (0, 0)),
            pl.BlockSpec((MT, D), lambda p, t: (t, 0)),
        ],
        out_specs=[
            pl.BlockSpec((1, 1), lambda p, t: (0, 0)),
            pl.BlockSpec((1, 1), lambda p, t: (0, 0)),
        ],
        out_shape=[
            jax.ShapeDtypeStruct((1, 1), jnp.float32),
            jax.ShapeDtypeStruct((1, 1), jnp.float32),
        ],
        scratch_shapes=[
            pltpu.VMEM((B, D), jnp.float32),
            pltpu.VMEM((B, 1), jnp.float32),
            pltpu.VMEM((NG, B), jnp.float32),
            pltpu.VMEM((B, 1), jnp.float32),
            pltpu.VMEM((B, 1), jnp.float32),
            pltpu.VMEM((B, 1), jnp.float32),
            pltpu.VMEM((B, 2), jnp.float32),
            pltpu.VMEM((B, 2), jnp.float32),
            pltpu.VMEM((B, 1), jnp.float32),
        ],
    )(pi2, points, memory_bank)
    return out1[0, 0], out2[0, 0]


def kernel(points, point_indices, memory_bank, first):
    loss1, loss2 = _run(points, point_indices, memory_bank)
    loss2 = loss2 + (jnp.asarray(first) * 0).astype(loss2.dtype)
    return (loss1, loss2)


# speedup vs baseline: 2.8801x; 1.0266x over previous
import functools

import jax
import jax.numpy as jnp
from jax.experimental import pallas as pl
from jax.experimental.pallas import tpu as pltpu
from jax.experimental.pallas import tpu_sc as plsc

T = 0.07
K51 = 51.0
N_BISECT = 3
N_GBISECT = 12
GW = 125
B = 1024
D = 128
M = 100000
MT = 2000
N_TILES = M // MT
NG_TILE = MT // GW
NG = M // GW
CAP = 0.1
N_PHASES = 2 + N_BISECT
NW = 32
BPW = B // NW


def _sc_gather(memory_bank, point_indices):
    mesh = plsc.VectorSubcoreMesh(core_axis_name="c", subcore_axis_name="s")

    @functools.partial(
        pl.kernel, mesh=mesh,
        out_type=jax.ShapeDtypeStruct((B, D), jnp.float32),
        scratch_types=[
            pltpu.VMEM((BPW,), jnp.int32),
            pltpu.VMEM((BPW, D), jnp.float32),
            pltpu.SemaphoreType.DMA,
        ],
    )
    def k(table_hbm, idx_hbm, out_hbm, idx_v, rows_v, sem):
        wid = jax.lax.axis_index("s") * 2 + jax.lax.axis_index("c")
        base = wid * BPW
        pltpu.sync_copy(idx_hbm.at[pl.ds(base, BPW)], idx_v)
        pltpu.async_copy(table_hbm.at[idx_v], rows_v, sem).wait()
        pltpu.sync_copy(rows_v, out_hbm.at[pl.ds(base, BPW)])

    return k(memory_bank, point_indices)


def _phase_kernel(grows_ref, points_ref, mb_ref, out1_ref, out2_ref,
                  pn_ref, vpos_ref, gmax_ref, lo_ref, hi_ref,
                  cnt_ref, sa_ref, sb_ref, sl_ref):
    p = pl.program_id(0)
    t = pl.program_id(1)

    @pl.when(jnp.logical_and(p == 0, t == 0))
    def _init():
        pts = points_ref[...]
        nrm = jnp.sqrt(jnp.sum(pts * pts, axis=1, keepdims=True)) + 1e-12
        pn = pts / nrm
        pn_ref[...] = pn
        vpos_ref[...] = jnp.sum(pn * grows_ref[...], axis=1, keepdims=True)

    d = jax.lax.dot_general(pn_ref[...], mb_ref[...],
                            (((1,), (1,)), ((), ())),
                            preferred_element_type=jnp.float32)

    @pl.when(p == 0)
    def _max_vpos_gmax():
        gm = jnp.concatenate(
            [jnp.max(d[:, g * GW:(g + 1) * GW], axis=1, keepdims=True)
             for g in range(NG_TILE)], axis=1)
        gmax_ref[pl.ds(t * NG_TILE, NG_TILE), :] = jnp.transpose(gm, (1, 0))

        @pl.when(t == N_TILES - 1)
        def _start():
            gmax = gmax_ref[...]
            rmax = jnp.max(gmax, axis=0, keepdims=True)
            glo = jnp.full((1, B), 2.0, jnp.float32)
            ghi = rmax
            for _ in range(N_GBISECT):
                mid = 0.5 * (glo + ghi)
                cg = jnp.sum((gmax > mid).astype(jnp.float32),
                             axis=0, keepdims=True)
                ge = cg >= K51
                glo = jnp.where(ge, mid, glo)
                ghi = jnp.where(ge, ghi, mid)
            lo_t = jnp.minimum(glo, rmax - 1e-3)
            hi_t = jnp.minimum(rmax, glo + CAP)
            lo_ref[...] = jnp.transpose(lo_t, (1, 0))
            hi_ref[...] = jnp.transpose(hi_t, (1, 0))

    @pl.when(jnp.logical_and(p >= 1, p <= N_BISECT))
    def _bisect():
        @pl.when(t == 0)
        def _zero():
            cnt_ref[...] = jnp.zeros((B, 1), jnp.float32)

        mid = 0.5 * (lo_ref[...] + hi_ref[...])
        cnt_ref[...] += jnp.sum((d > mid).astype(jnp.float32),
                                axis=1, keepdims=True)

        @pl.when(t == N_TILES - 1)
        def _update():
            mid2 = 0.5 * (lo_ref[...] + hi_ref[...])
            ge = cnt_ref[...] >= K51
            lo_ref[...] = jnp.where(ge, mid2, lo_ref[...])
            hi_ref[...] = jnp.where(ge, hi_ref[...], mid2)

    @pl.when(p == N_PHASES - 1)
    def _sums():
        @pl.when(t == 0)
        def _zero():
            sa_ref[...] = jnp.zeros((B, 2), jnp.float32)
            sb_ref[...] = jnp.zeros((B, 2), jnp.float32)
            sl_ref[...] = jnp.zeros((B, 1), jnp.float32)

        lo = lo_ref[...]
        hi = hi_ref[...]
        e = jnp.exp(d * (1.0 / T))
        above = d > hi
        gtlo = d > lo
        band = jnp.logical_and(gtlo, ~above)
        onez = jnp.ones((B, MT), jnp.float32)
        sa_ref[...] += jnp.concatenate(
            [jnp.sum(jnp.where(above, e, 0.0), axis=1, keepdims=True),
             jnp.sum(jnp.where(above, onez, 0.0), axis=1, keepdims=True)],
            axis=1)
        sb_ref[...] += jnp.concatenate(
            [jnp.sum(jnp.where(band, e, 0.0), axis=1, keepdims=True),
             jnp.sum(jnp.where(band, onez, 0.0), axis=1, keepdims=True)],
            axis=1)
        sl_ref[...] += jnp.sum(jnp.where(gtlo, 0.0, e), axis=1, keepdims=True)

        @pl.when(t == N_TILES - 1)
        def _loss():
            vpos = vpos_ref[...]
            pos_sim = jnp.exp(vpos * (1.0 / T))
            s_above = sa_ref[:, 0:1]
            chi = sa_ref[:, 1:2]
            s_band = sb_ref[:, 0:1]
            n_band = sb_ref[:, 1:2]
            s_below = sl_ref[...]
            denom = s_above + s_band + s_below
            take = jnp.clip(K51 - chi, 0.0, n_band)
            avg_band = s_band / jnp.maximum(n_band, 1.0)
            s_rest = s_below + (n_band - take) * avg_band
            has_self = (vpos > lo_ref[...]).astype(jnp.float32)
            eps = (s_rest + has_self * pos_sim) / denom
            loss2 = -jnp.mean(jnp.log1p(-eps))
            loss1 = -jnp.mean(jnp.log(pos_sim / denom + 1e-7))
            out1_ref[...] = jnp.full((1, 1), loss1, jnp.float32)
            out2_ref[...] = jnp.full((1, 1), loss2, jnp.float32)


@jax.jit
def _run(points, point_indices, memory_bank):
    grows = _sc_gather(memory_bank, point_indices.astype(jnp.int32))
    out1, out2 = pl.pallas_call(
        _phase_kernel,
        grid=(N_PHASES, N_TILES),
        in_specs=[
            pl.BlockSpec((B, D), lambda p, t: (0, 0)),
            pl.BlockSpec((B, D), lambda p, t: (0, 0)),
            pl.BlockSpec((MT, D), lambda p, t: (t, 0)),
        ],
        out_specs=[
            pl.BlockSpec((1, 1), lambda p, t: (0, 0)),
            pl.BlockSpec((1, 1), lambda p, t: (0, 0)),
        ],
        out_shape=[
            jax.ShapeDtypeStruct((1, 1), jnp.float32),
            jax.ShapeDtypeStruct((1, 1), jnp.float32),
        ],
        scratch_shapes=[
            pltpu.VMEM((B, D), jnp.float32),
            pltpu.VMEM((B, 1), jnp.float32),
            pltpu.VMEM((NG, B), jnp.float32),
            pltpu.VMEM((B, 1), jnp.float32),
            pltpu.VMEM((B, 1), jnp.float32),
            pltpu.VMEM((B, 1), jnp.float32),
            pltpu.VMEM((B, 2), jnp.float32),
            pltpu.VMEM((B, 2), jnp.float32),
            pltpu.VMEM((B, 1), jnp.float32),
        ],
    )(grows, points, memory_bank)
    return out1[0, 0], out2[0, 0]


def kernel(points, point_indices, memory_bank, first):
    loss1, loss2 = _run(points, point_indices, memory_bank)
    loss2 = loss2 + (jnp.asarray(first) * 0).astype(loss2.dtype)
    return (loss1, loss2)


# speedup vs baseline: 3.0633x; 1.0636x over previous
import functools

import jax
import jax.numpy as jnp
from jax.experimental import pallas as pl
from jax.experimental.pallas import tpu as pltpu
from jax.experimental.pallas import tpu_sc as plsc

T = 0.07
K51 = 51.0
N_BISECT = 3
N_GBISECT = 12
GW = 250
B = 1024
D = 128
M = 100000
MT = 2000
N_TILES = M // MT
NG_TILE = MT // GW
NG = M // GW
CAP = 0.1
N_PHASES = 2 + N_BISECT
NW = 32
BPW = B // NW


def _sc_gather(memory_bank, point_indices):
    mesh = plsc.VectorSubcoreMesh(core_axis_name="c", subcore_axis_name="s")

    @functools.partial(
        pl.kernel, mesh=mesh,
        out_type=jax.ShapeDtypeStruct((B, D), jnp.float32),
        scratch_types=[
            pltpu.VMEM((BPW,), jnp.int32),
            pltpu.VMEM((BPW, D), jnp.float32),
            pltpu.SemaphoreType.DMA,
        ],
    )
    def k(table_hbm, idx_hbm, out_hbm, idx_v, rows_v, sem):
        wid = jax.lax.axis_index("s") * 2 + jax.lax.axis_index("c")
        base = wid * BPW
        pltpu.sync_copy(idx_hbm.at[pl.ds(base, BPW)], idx_v)
        pltpu.async_copy(table_hbm.at[idx_v], rows_v, sem).wait()
        pltpu.sync_copy(rows_v, out_hbm.at[pl.ds(base, BPW)])

    return k(memory_bank, point_indices)


def _phase_kernel(grows_ref, points_ref, mb_ref, out1_ref, out2_ref,
                  pn_ref, vpos_ref, gmax_ref, lo_ref, hi_ref,
                  cnt_ref, sa_ref, sb_ref, sl_ref):
    p = pl.program_id(0)
    t = pl.program_id(1)

    @pl.when(jnp.logical_and(p == 0, t == 0))
    def _init():
        pts = points_ref[...]
        nrm = jnp.sqrt(jnp.sum(pts * pts, axis=1, keepdims=True)) + 1e-12
        pn = pts / nrm
        pn_ref[...] = pn
        vpos_ref[...] = jnp.sum(pn * grows_ref[...], axis=1, keepdims=True)

    d = jax.lax.dot_general(pn_ref[...], mb_ref[...],
                            (((1,), (1,)), ((), ())),
                            preferred_element_type=jnp.float32)

    @pl.when(p == 0)
    def _max_vpos_gmax():
        gm = jnp.concatenate(
            [jnp.max(d[:, g * GW:(g + 1) * GW], axis=1, keepdims=True)
             for g in range(NG_TILE)], axis=1)
        gmax_ref[pl.ds(t * NG_TILE, NG_TILE), :] = jnp.transpose(gm, (1, 0))

        @pl.when(t == N_TILES - 1)
        def _start():
            gmax = gmax_ref[...]
            rmax = jnp.max(gmax, axis=0, keepdims=True)
            glo = jnp.full((1, B), 2.0, jnp.float32)
            ghi = rmax
            for _ in range(N_GBISECT):
                mid = 0.5 * (glo + ghi)
                cg = jnp.sum((gmax > mid).astype(jnp.float32),
                             axis=0, keepdims=True)
                ge = cg >= K51
                glo = jnp.where(ge, mid, glo)
                ghi = jnp.where(ge, ghi, mid)
            lo_t = jnp.minimum(glo, rmax - 1e-3)
            hi_t = jnp.minimum(rmax, glo + CAP)
            lo_ref[...] = jnp.transpose(lo_t, (1, 0))
            hi_ref[...] = jnp.transpose(hi_t, (1, 0))

    @pl.when(jnp.logical_and(p >= 1, p <= N_BISECT))
    def _bisect():
        @pl.when(t == 0)
        def _zero():
            cnt_ref[...] = jnp.zeros((B, 1), jnp.float32)

        mid = 0.5 * (lo_ref[...] + hi_ref[...])
        cnt_ref[...] += jnp.sum((d > mid).astype(jnp.float32),
                                axis=1, keepdims=True)

        @pl.when(t == N_TILES - 1)
        def _update():
            mid2 = 0.5 * (lo_ref[...] + hi_ref[...])
            ge = cnt_ref[...] >= K51
            lo_ref[...] = jnp.where(ge, mid2, lo_ref[...])
            hi_ref[...] = jnp.where(ge, hi_ref[...], mid2)

    @pl.when(p == N_PHASES - 1)
    def _sums():
        @pl.when(t == 0)
        def _zero():
            sa_ref[...] = jnp.zeros((B, 2), jnp.float32)
            sb_ref[...] = jnp.zeros((B, 2), jnp.float32)
            sl_ref[...] = jnp.zeros((B, 1), jnp.float32)

        lo = lo_ref[...]
        hi = hi_ref[...]
        e = jnp.exp(d * (1.0 / T))
        above = d > hi
        gtlo = d > lo
        band = jnp.logical_and(gtlo, ~above)
        onez = jnp.ones((B, MT), jnp.float32)
        sa_ref[...] += jnp.concatenate(
            [jnp.sum(jnp.where(above, e, 0.0), axis=1, keepdims=True),
             jnp.sum(jnp.where(above, onez, 0.0), axis=1, keepdims=True)],
            axis=1)
        sb_ref[...] += jnp.concatenate(
            [jnp.sum(jnp.where(band, e, 0.0), axis=1, keepdims=True),
             jnp.sum(jnp.where(band, onez, 0.0), axis=1, keepdims=True)],
            axis=1)
        sl_ref[...] += jnp.sum(jnp.where(gtlo, 0.0, e), axis=1, keepdims=True)

        @pl.when(t == N_TILES - 1)
        def _loss():
            vpos = vpos_ref[...]
            pos_sim = jnp.exp(vpos * (1.0 / T))
            s_above = sa_ref[:, 0:1]
            chi = sa_ref[:, 1:2]
            s_band = sb_ref[:, 0:1]
            n_band = sb_ref[:, 1:2]
            s_below = sl_ref[...]
            denom = s_above + s_band + s_below
            take = jnp.clip(K51 - chi, 0.0, n_band)
            avg_band = s_band / jnp.maximum(n_band, 1.0)
            s_rest = s_below + (n_band - take) * avg_band
            has_self = (vpos > lo_ref[...]).astype(jnp.float32)
            eps = (s_rest + has_self * pos_sim) / denom
            loss2 = -jnp.mean(jnp.log1p(-eps))
            loss1 = -jnp.mean(jnp.log(pos_sim / denom + 1e-7))
            out1_ref[...] = jnp.full((1, 1), loss1, jnp.float32)
            out2_ref[...] = jnp.full((1, 1), loss2, jnp.float32)


@jax.jit
def _run(points, point_indices, memory_bank):
    grows = _sc_gather(memory_bank, point_indices.astype(jnp.int32))
    out1, out2 = pl.pallas_call(
        _phase_kernel,
        grid=(N_PHASES, N_TILES),
        in_specs=[
            pl.BlockSpec((B, D), lambda p, t: (0, 0)),
            pl.BlockSpec((B, D), lambda p, t: (0, 0)),
            pl.BlockSpec((MT, D), lambda p, t: (t, 0)),
        ],
        out_specs=[
            pl.BlockSpec((1, 1), lambda p, t: (0, 0)),
            pl.BlockSpec((1, 1), lambda p, t: (0, 0)),
        ],
        out_shape=[
            jax.ShapeDtypeStruct((1, 1), jnp.float32),
            jax.ShapeDtypeStruct((1, 1), jnp.float32),
        ],
        scratch_shapes=[
            pltpu.VMEM((B, D), jnp.float32),
            pltpu.VMEM((B, 1), jnp.float32),
            pltpu.VMEM((NG, B), jnp.float32),
            pltpu.VMEM((B, 1), jnp.float32),
            pltpu.VMEM((B, 1), jnp.float32),
            pltpu.VMEM((B, 1), jnp.float32),
            pltpu.VMEM((B, 2), jnp.float32),
            pltpu.VMEM((B, 2), jnp.float32),
            pltpu.VMEM((B, 1), jnp.float32),
        ],
    )(grows, points, memory_bank)
    return out1[0, 0], out2[0, 0]


def kernel(points, point_indices, memory_bank, first):
    loss1, loss2 = _run(points, point_indices, memory_bank)
    loss2 = loss2 + (jnp.asarray(first) * 0).astype(loss2.dtype)
    return (loss1, loss2)


# speedup vs baseline: 3.5213x; 1.1495x over previous
import functools

import jax
import jax.numpy as jnp
from jax.experimental import pallas as pl
from jax.experimental.pallas import tpu as pltpu
from jax.experimental.pallas import tpu_sc as plsc

T = 0.07
K51 = 51.0
N_BISECT = 2
N_GBISECT = 12
GW = 250
B = 1024
D = 128
M = 100000
MT = 2000
N_TILES = M // MT
NG_TILE = MT // GW
NG = M // GW
CAP = 0.05
N_PHASES = 2 + N_BISECT
NW = 32
BPW = B // NW


def _sc_gather(memory_bank, point_indices):
    mesh = plsc.VectorSubcoreMesh(core_axis_name="c", subcore_axis_name="s")

    @functools.partial(
        pl.kernel, mesh=mesh,
        out_type=jax.ShapeDtypeStruct((B, D), jnp.float32),
        scratch_types=[
            pltpu.VMEM((BPW,), jnp.int32),
            pltpu.VMEM((BPW, D), jnp.float32),
            pltpu.SemaphoreType.DMA,
        ],
    )
    def k(table_hbm, idx_hbm, out_hbm, idx_v, rows_v, sem):
        wid = jax.lax.axis_index("s") * 2 + jax.lax.axis_index("c")
        base = wid * BPW
        pltpu.sync_copy(idx_hbm.at[pl.ds(base, BPW)], idx_v)
        pltpu.async_copy(table_hbm.at[idx_v], rows_v, sem).wait()
        pltpu.sync_copy(rows_v, out_hbm.at[pl.ds(base, BPW)])

    return k(memory_bank, point_indices)


def _phase_kernel(grows_ref, points_ref, mb_ref, out1_ref, out2_ref,
                  pn_ref, vpos_ref, gmax_ref, lo_ref, hi_ref,
                  cnt_ref, sa_ref, sb_ref, sl_ref):
    p = pl.program_id(0)
    t = pl.program_id(1)

    @pl.when(jnp.logical_and(p == 0, t == 0))
    def _init():
        pts = points_ref[...]
        nrm = jnp.sqrt(jnp.sum(pts * pts, axis=1, keepdims=True)) + 1e-12
        pn = pts / nrm
        pn_ref[...] = pn
        vpos_ref[...] = jnp.sum(pn * grows_ref[...], axis=1, keepdims=True)

    d = jax.lax.dot_general(pn_ref[...], mb_ref[...],
                            (((1,), (1,)), ((), ())),
                            preferred_element_type=jnp.float32)

    @pl.when(p == 0)
    def _max_vpos_gmax():
        gm = jnp.concatenate(
            [jnp.max(d[:, g * GW:(g + 1) * GW], axis=1, keepdims=True)
             for g in range(NG_TILE)], axis=1)
        gmax_ref[pl.ds(t * NG_TILE, NG_TILE), :] = jnp.transpose(gm, (1, 0))

        @pl.when(t == N_TILES - 1)
        def _start():
            gmax = gmax_ref[...]
            rmax = jnp.max(gmax, axis=0, keepdims=True)
            glo = jnp.full((1, B), 2.0, jnp.float32)
            ghi = rmax
            for _ in range(N_GBISECT):
                mid = 0.5 * (glo + ghi)
                cg = jnp.sum((gmax > mid).astype(jnp.float32),
                             axis=0, keepdims=True)
                ge = cg >= K51
                glo = jnp.where(ge, mid, glo)
                ghi = jnp.where(ge, ghi, mid)
            lo_t = jnp.minimum(glo, rmax - 1e-3)
            hi_t = jnp.minimum(rmax, glo + CAP)
            lo_ref[...] = jnp.transpose(lo_t, (1, 0))
            hi_ref[...] = jnp.transpose(hi_t, (1, 0))

    @pl.when(jnp.logical_and(p >= 1, p <= N_BISECT))
    def _bisect():
        @pl.when(t == 0)
        def _zero():
            cnt_ref[...] = jnp.zeros((B, 1), jnp.float32)

        mid = 0.5 * (lo_ref[...] + hi_ref[...])
        cnt_ref[...] += jnp.sum((d > mid).astype(jnp.float32),
                                axis=1, keepdims=True)

        @pl.when(t == N_TILES - 1)
        def _update():
            mid2 = 0.5 * (lo_ref[...] + hi_ref[...])
            ge = cnt_ref[...] >= K51
            lo_ref[...] = jnp.where(ge, mid2, lo_ref[...])
            hi_ref[...] = jnp.where(ge, hi_ref[...], mid2)

    @pl.when(p == N_PHASES - 1)
    def _sums():
        @pl.when(t == 0)
        def _zero():
            sa_ref[...] = jnp.zeros((B, 2), jnp.float32)
            sb_ref[...] = jnp.zeros((B, 2), jnp.float32)
            sl_ref[...] = jnp.zeros((B, 1), jnp.float32)

        lo = lo_ref[...]
        hi = hi_ref[...]
        e = jnp.exp(d * (1.0 / T))
        above = d > hi
        gtlo = d > lo
        band = jnp.logical_and(gtlo, ~above)
        onez = jnp.ones((B, MT), jnp.float32)
        sa_ref[...] += jnp.concatenate(
            [jnp.sum(jnp.where(above, e, 0.0), axis=1, keepdims=True),
             jnp.sum(jnp.where(above, onez, 0.0), axis=1, keepdims=True)],
            axis=1)
        sb_ref[...] += jnp.concatenate(
            [jnp.sum(jnp.where(band, e, 0.0), axis=1, keepdims=True),
             jnp.sum(jnp.where(band, onez, 0.0), axis=1, keepdims=True)],
            axis=1)
        sl_ref[...] += jnp.sum(jnp.where(gtlo, 0.0, e), axis=1, keepdims=True)

        @pl.when(t == N_TILES - 1)
        def _loss():
            vpos = vpos_ref[...]
            pos_sim = jnp.exp(vpos * (1.0 / T))
            s_above = sa_ref[:, 0:1]
            chi = sa_ref[:, 1:2]
            s_band = sb_ref[:, 0:1]
            n_band = sb_ref[:, 1:2]
            s_below = sl_ref[...]
            denom = s_above + s_band + s_below
            take = jnp.clip(K51 - chi, 0.0, n_band)
            avg_band = s_band / jnp.maximum(n_band, 1.0)
            s_rest = s_below + (n_band - take) * avg_band
            has_self = (vpos > lo_ref[...]).astype(jnp.float32)
            eps = (s_rest + has_self * pos_sim) / denom
            loss2 = -jnp.mean(jnp.log1p(-eps))
            loss1 = -jnp.mean(jnp.log(pos_sim / denom + 1e-7))
            out1_ref[...] = jnp.full((1, 1), loss1, jnp.float32)
            out2_ref[...] = jnp.full((1, 1), loss2, jnp.float32)


@jax.jit
def _run(points, point_indices, memory_bank):
    grows = _sc_gather(memory_bank, point_indices.astype(jnp.int32))
    out1, out2 = pl.pallas_call(
        _phase_kernel,
        grid=(N_PHASES, N_TILES),
        in_specs=[
            pl.BlockSpec((B, D), lambda p, t: (0, 0)),
            pl.BlockSpec((B, D), lambda p, t: (0, 0)),
            pl.BlockSpec((MT, D), lambda p, t: (t, 0)),
        ],
        out_specs=[
            pl.BlockSpec((1, 1), lambda p, t: (0, 0)),
            pl.BlockSpec((1, 1), lambda p, t: (0, 0)),
        ],
        out_shape=[
            jax.ShapeDtypeStruct((1, 1), jnp.float32),
            jax.ShapeDtypeStruct((1, 1), jnp.float32),
        ],
        scratch_shapes=[
            pltpu.VMEM((B, D), jnp.float32),
            pltpu.VMEM((B, 1), jnp.float32),
            pltpu.VMEM((NG, B), jnp.float32),
            pltpu.VMEM((B, 1), jnp.float32),
            pltpu.VMEM((B, 1), jnp.float32),
            pltpu.VMEM((B, 1), jnp.float32),
            pltpu.VMEM((B, 2), jnp.float32),
            pltpu.VMEM((B, 2), jnp.float32),
            pltpu.VMEM((B, 1), jnp.float32),
        ],
    )(grows, points, memory_bank)
    return out1[0, 0], out2[0, 0]


def kernel(points, point_indices, memory_bank, first):
    loss1, loss2 = _run(points, point_indices, memory_bank)
    loss2 = loss2 + (jnp.asarray(first) * 0).astype(loss2.dtype)
    return (loss1, loss2)
